# Initial kernel scaffold; baseline (speedup 1.0000x reference)
import functools

import jax
import jax.numpy as jnp
from jax import lax
from jax.experimental import pallas as pl
from jax.experimental.pallas import tpu as pltpu
from jax.experimental.pallas import tpu_sc as plsc

_MESH = plsc.VectorSubcoreMesh(core_axis_name="c", subcore_axis_name="s")
_NW = 32
_K = 80


def _node_transforms(h, WA, bA, WB, bB, WD, bD, WE, bE):
    n, d = h.shape
    bm = 2000
    grid = (n // bm,)
    x_spec = pl.BlockSpec((bm, d), lambda i: (i, 0))
    w_spec = pl.BlockSpec((d, d), lambda i: (0, 0))
    b_spec = pl.BlockSpec((1, d), lambda i: (0, 0))

    def body(x_ref, wa, ba, wb, bb, wd, bd, we, be_, oa, ob, od, oe):
        x = x_ref[...]
        oa[...] = jnp.dot(x, wa[...], preferred_element_type=jnp.float32) + ba[...]
        ob[...] = jnp.dot(x, wb[...], preferred_element_type=jnp.float32) + bb[...]
        od[...] = jnp.dot(x, wd[...], preferred_element_type=jnp.float32) + bd[...]
        oe[...] = jnp.dot(x, we[...], preferred_element_type=jnp.float32) + be_[...]

    return pl.pallas_call(
        body,
        grid=grid,
        in_specs=[x_spec, w_spec, b_spec, w_spec, b_spec, w_spec, b_spec,
                  w_spec, b_spec],
        out_specs=[x_spec] * 4,
        out_shape=[jax.ShapeDtypeStruct((n, d), jnp.float32)] * 4,
    )(h, WA, bA.reshape(1, d), WB, bB.reshape(1, d), WD, bD.reshape(1, d),
      WE, bE.reshape(1, d))


def _sc_gather3(Dh, Eh, Bh, src, dst):
    n, d = Dh.shape
    ecount = src.shape[0]
    per_w = ecount // _NW

    @functools.partial(
        pl.kernel,
        mesh=_MESH,
        out_type=[jax.ShapeDtypeStruct((ecount, d), jnp.float32)] * 3,
        scratch_types=[
            pltpu.VMEM((_K,), jnp.int32),
            pltpu.VMEM((_K,), jnp.int32),
            pltpu.VMEM((_K, d), jnp.float32),
            pltpu.VMEM((_K, d), jnp.float32),
            pltpu.VMEM((_K, d), jnp.float32),
            pltpu.SemaphoreType.DMA,
            pltpu.SemaphoreType.DMA,
            pltpu.SemaphoreType.DMA,
        ],
    )
    def gk(dh_h, eh_h, bh_h, src_h, dst_h, od_h, oe_h, ob_h,
           si_v, di_v, rd_v, re_v, rb_v, s0, s1, s2):
        wid = lax.axis_index("s") * 2 + lax.axis_index("c")
        base = wid * per_w

        @pl.loop(0, per_w // _K)
        def _(ci):
            b = base + ci * _K
            pltpu.sync_copy(src_h.at[pl.ds(b, _K)], si_v)
            pltpu.sync_copy(dst_h.at[pl.ds(b, _K)], di_v)
            c0 = pltpu.async_copy(dh_h.at[si_v], rd_v, s0)
            c1 = pltpu.async_copy(eh_h.at[di_v], re_v, s1)
            c2 = pltpu.async_copy(bh_h.at[si_v], rb_v, s2)
            c0.wait()
            c1.wait()
            c2.wait()
            pltpu.sync_copy(rd_v, od_h.at[pl.ds(b, _K)])
            pltpu.sync_copy(re_v, oe_h.at[pl.ds(b, _K)])
            pltpu.sync_copy(rb_v, ob_h.at[pl.ds(b, _K)])

    return gk(Dh, Eh, Bh, src, dst)


def _edge_math(e, dsrc, edst, bsrc, WC, bC):
    m, d = e.shape
    bm = 2000
    grid = (m // bm,)
    x_spec = pl.BlockSpec((bm, d), lambda i: (i, 0))

    def body(e_ref, ds_ref, ed_ref, bs_ref, wc, bc, ms_ref, eo_ref):
        ev = e_ref[...]
        ce = jnp.dot(ev, wc[...], preferred_element_type=jnp.float32) + bc[...]
        e_new = ds_ref[...] + ed_ref[...] + ce
        sig = jax.nn.sigmoid(e_new)
        ms_ref[0] = bs_ref[...] * sig
        ms_ref[1] = sig
        eo_ref[...] = ev + e_new * sig

    return pl.pallas_call(
        body,
        grid=grid,
        in_specs=[x_spec, x_spec, x_spec, x_spec,
                  pl.BlockSpec((d, d), lambda i: (0, 0)),
                  pl.BlockSpec((1, d), lambda i: (0, 0))],
        out_specs=[pl.BlockSpec((2, bm, d), lambda i: (0, i, 0)), x_spec],
        out_shape=[jax.ShapeDtypeStruct((2, m, d), jnp.float32),
                   jax.ShapeDtypeStruct((m, d), jnp.float32)],
    )(e, dsrc, edst, bsrc, WC, bC.reshape(1, d))


def _sc_scatter_add(ms_flat, dst, zeros):
    e2, d = ms_flat.shape
    ecount = e2 // 2
    n = zeros.shape[0]
    per_s = ecount // 16

    @functools.partial(
        pl.kernel,
        mesh=_MESH,
        out_type=jax.ShapeDtypeStruct((2 * n, d), jnp.float32),
        scratch_types=[
            pltpu.VMEM((_K,), jnp.int32),
            pltpu.VMEM((_K, d), jnp.float32),
            pltpu.VMEM_SHARED((n, d), jnp.float32),
        ],
    )
    def sk(ms_h, dst_h, z_h, o_h, idx_v, rows_v, acc):
        core = lax.axis_index("c")
        sid = lax.axis_index("s")

        @pl.when(sid == 0)
        def _():
            pltpu.sync_copy(z_h, acc)

        plsc.subcore_barrier()
        ebase = core * ecount + sid * per_s
        ibase = sid * per_s

        @pl.loop(0, per_s // _K)
        def _(ci):
            pltpu.sync_copy(dst_h.at[pl.ds(ibase + ci * _K, _K)], idx_v)
            pltpu.sync_copy(ms_h.at[pl.ds(ebase + ci * _K, _K)], rows_v)
            pltpu.sync_copy(rows_v, acc.at[idx_v], add=True)

        plsc.subcore_barrier()
        nsl = n // 10

        @pl.when(sid < 10)
        def _():
            pltpu.sync_copy(acc.at[pl.ds(sid * nsl, nsl)],
                            o_h.at[pl.ds(core * n + sid * nsl, nsl)])

    return sk(ms_flat, dst, zeros)


def _node_update(h, Ah, acc):
    n, d = h.shape
    bm = 2000
    grid = (n // bm,)
    x_spec = pl.BlockSpec((bm, d), lambda i: (i, 0))

    def body(h_ref, a_ref, acc_ref, o_ref):
        hn = a_ref[...] + acc_ref[0] / (acc_ref[1] + 1e-6)
        o_ref[...] = h_ref[...] + hn * jax.nn.sigmoid(hn)

    return pl.pallas_call(
        body,
        grid=grid,
        in_specs=[x_spec, x_spec,
                  pl.BlockSpec((2, bm, d), lambda i: (0, i, 0))],
        out_specs=x_spec,
        out_shape=jax.ShapeDtypeStruct((n, d), jnp.float32),
    )(h, Ah, acc)


def kernel(h, e, edge_index, WA, bA, WB, bB, WC, bC, WD, bD, WE, bE):
    src = edge_index[0]
    dst = edge_index[1]
    n, d = h.shape
    ecount = e.shape[0]
    Ah, Bh, Dh, Eh = _node_transforms(h, WA, bA, WB, bB, WD, bD, WE, bE)
    dsrc, edst, bsrc = _sc_gather3(Dh, Eh, Bh, src, dst)
    ms, e_out = _edge_math(e, dsrc, edst, bsrc, WC, bC)
    zeros = jnp.zeros((n, d), jnp.float32)
    acc_flat = _sc_scatter_add(ms.reshape(2 * ecount, d), dst, zeros)
    h_out = _node_update(h, Ah, acc_flat.reshape(2, n, d))
    return (h_out, e_out)


# speedup vs baseline: 2.9535x; 2.9535x over previous
import functools

import jax
import jax.numpy as jnp
from jax import lax
from jax.experimental import pallas as pl
from jax.experimental.pallas import tpu as pltpu
from jax.experimental.pallas import tpu_sc as plsc

def _sc_mesh():
    return plsc.VectorSubcoreMesh(core_axis_name="c", subcore_axis_name="s")


_NW = 32
_K = 80


def _node_transforms(h, WA, bA, WB, bB, WD, bD, WE, bE):
    n, d = h.shape
    bm = 2000
    grid = (n // bm,)
    x_spec = pl.BlockSpec((bm, d), lambda i: (i, 0))
    w_spec = pl.BlockSpec((d, d), lambda i: (0, 0))
    b_spec = pl.BlockSpec((1, d), lambda i: (0, 0))

    def body(x_ref, wa, ba, wb, bb, wd, bd, we, be_, oa, ob, od, oe):
        x = x_ref[...]
        oa[...] = jnp.dot(x, wa[...], preferred_element_type=jnp.float32) + ba[...]
        ob[...] = jnp.dot(x, wb[...], preferred_element_type=jnp.float32) + bb[...]
        od[...] = jnp.dot(x, wd[...], preferred_element_type=jnp.float32) + bd[...]
        oe[...] = jnp.dot(x, we[...], preferred_element_type=jnp.float32) + be_[...]

    return pl.pallas_call(
        body,
        grid=grid,
        in_specs=[x_spec, w_spec, b_spec, w_spec, b_spec, w_spec, b_spec,
                  w_spec, b_spec],
        out_specs=[x_spec] * 4,
        out_shape=[jax.ShapeDtypeStruct((n, d), jnp.float32)] * 4,
    )(h, WA, bA.reshape(1, d), WB, bB.reshape(1, d), WD, bD.reshape(1, d),
      WE, bE.reshape(1, d))


def _sc_gather3(Dh, Eh, Bh, src, dst):
    n, d = Dh.shape
    ecount = src.shape[0]
    per_w = ecount // _NW

    @functools.partial(
        pl.kernel,
        mesh=_sc_mesh(),
        out_type=[jax.ShapeDtypeStruct((ecount, d), jnp.float32)] * 3,
        scratch_types=[
            pltpu.VMEM((_K,), jnp.int32),
            pltpu.VMEM((_K,), jnp.int32),
            pltpu.VMEM((_K, d), jnp.float32),
            pltpu.VMEM((_K, d), jnp.float32),
            pltpu.VMEM((_K, d), jnp.float32),
            pltpu.SemaphoreType.DMA,
            pltpu.SemaphoreType.DMA,
            pltpu.SemaphoreType.DMA,
        ],
    )
    def gk(dh_h, eh_h, bh_h, src_h, dst_h, od_h, oe_h, ob_h,
           si_v, di_v, rd_v, re_v, rb_v, s0, s1, s2):
        wid = lax.axis_index("s") * 2 + lax.axis_index("c")
        base = wid * per_w

        @pl.loop(0, per_w // _K)
        def _(ci):
            b = base + ci * _K
            pltpu.sync_copy(src_h.at[pl.ds(b, _K)], si_v)
            pltpu.sync_copy(dst_h.at[pl.ds(b, _K)], di_v)
            c0 = pltpu.async_copy(dh_h.at[si_v], rd_v, s0)
            c1 = pltpu.async_copy(eh_h.at[di_v], re_v, s1)
            c2 = pltpu.async_copy(bh_h.at[si_v], rb_v, s2)
            c0.wait()
            c1.wait()
            c2.wait()
            pltpu.sync_copy(rd_v, od_h.at[pl.ds(b, _K)])
            pltpu.sync_copy(re_v, oe_h.at[pl.ds(b, _K)])
            pltpu.sync_copy(rb_v, ob_h.at[pl.ds(b, _K)])

    return gk(Dh, Eh, Bh, src, dst)


def _edge_math(e, dsrc, edst, bsrc, WC, bC):
    m, d = e.shape
    bm = 2000
    grid = (m // bm,)
    x_spec = pl.BlockSpec((bm, d), lambda i: (i, 0))

    def body(e_ref, ds_ref, ed_ref, bs_ref, wc, bc, ms_ref, eo_ref):
        ev = e_ref[...]
        ce = jnp.dot(ev, wc[...], preferred_element_type=jnp.float32) + bc[...]
        e_new = ds_ref[...] + ed_ref[...] + ce
        sig = jax.nn.sigmoid(e_new)
        ms_ref[0] = bs_ref[...] * sig
        ms_ref[1] = sig
        eo_ref[...] = ev + e_new * sig

    return pl.pallas_call(
        body,
        grid=grid,
        in_specs=[x_spec, x_spec, x_spec, x_spec,
                  pl.BlockSpec((d, d), lambda i: (0, 0)),
                  pl.BlockSpec((1, d), lambda i: (0, 0))],
        out_specs=[pl.BlockSpec((2, bm, d), lambda i: (0, i, 0)), x_spec],
        out_shape=[jax.ShapeDtypeStruct((2, m, d), jnp.float32),
                   jax.ShapeDtypeStruct((m, d), jnp.float32)],
    )(e, dsrc, edst, bsrc, WC, bC.reshape(1, d))


def _sc_scatter_add(ms_flat, dst, zeros):
    e2, d = ms_flat.shape
    ecount = e2 // 2
    n = zeros.shape[0]
    per_s = ecount // 16

    @functools.partial(
        pl.kernel,
        mesh=_sc_mesh(),
        out_type=jax.ShapeDtypeStruct((2 * n, d), jnp.float32),
        scratch_types=[
            pltpu.VMEM((_K,), jnp.int32),
            pltpu.VMEM((_K, d), jnp.float32),
            pltpu.VMEM_SHARED((n, d), jnp.float32),
        ],
    )
    def sk(ms_h, dst_h, z_h, o_h, idx_v, rows_v, acc):
        core = lax.axis_index("c")
        sid = lax.axis_index("s")

        @pl.when(sid == 0)
        def _():
            pltpu.sync_copy(z_h, acc)

        plsc.subcore_barrier()
        ebase = core * ecount + sid * per_s
        ibase = sid * per_s

        @pl.loop(0, per_s // _K)
        def _(ci):
            pltpu.sync_copy(dst_h.at[pl.ds(ibase + ci * _K, _K)], idx_v)
            pltpu.sync_copy(ms_h.at[pl.ds(ebase + ci * _K, _K)], rows_v)
            pltpu.sync_copy(rows_v, acc.at[idx_v], add=True)

        plsc.subcore_barrier()
        nsl = n // 10

        @pl.when(sid < 10)
        def _():
            pltpu.sync_copy(acc.at[pl.ds(sid * nsl, nsl)],
                            o_h.at[pl.ds(core * n + sid * nsl, nsl)])

    return sk(ms_flat, dst, zeros)


def _node_update(h, Ah, acc):
    n, d = h.shape
    bm = 2000
    grid = (n // bm,)
    x_spec = pl.BlockSpec((bm, d), lambda i: (i, 0))

    def body(h_ref, a_ref, acc_ref, o_ref):
        hn = a_ref[...] + acc_ref[0] / (acc_ref[1] + 1e-6)
        o_ref[...] = h_ref[...] + hn * jax.nn.sigmoid(hn)

    return pl.pallas_call(
        body,
        grid=grid,
        in_specs=[x_spec, x_spec,
                  pl.BlockSpec((2, bm, d), lambda i: (0, i, 0))],
        out_specs=x_spec,
        out_shape=jax.ShapeDtypeStruct((n, d), jnp.float32),
    )(h, Ah, acc)


def kernel(h, e, edge_index, WA, bA, WB, bB, WC, bC, WD, bD, WE, bE):
    src = edge_index[0]
    dst = edge_index[1]
    n, d = h.shape
    ecount = e.shape[0]
    Ah, Bh, Dh, Eh = _node_transforms(h, WA, bA, WB, bB, WD, bD, WE, bE)
    dsrc, edst, bsrc = _sc_gather3(Dh, Eh, Bh, src, dst)
    ms, e_out = _edge_math(e, dsrc, edst, bsrc, WC, bC)
    zeros = jnp.zeros((n, d), jnp.float32)
    acc_flat = _sc_scatter_add(ms.reshape(2 * ecount, d), dst, zeros)
    h_out = _node_update(h, Ah, acc_flat.reshape(2, n, d))
    return (h_out, e_out)


# speedup vs baseline: 4.2790x; 1.4488x over previous
import functools

import jax
import jax.numpy as jnp
from jax import lax
from jax.experimental import pallas as pl
from jax.experimental.pallas import tpu as pltpu
from jax.experimental.pallas import tpu_sc as plsc


def _sc_mesh():
    return plsc.VectorSubcoreMesh(core_axis_name="c", subcore_axis_name="s")


_NW = 32
_K = 40
_G = 5


def _node_transforms(h, WA, bA, WB, bB, WD, bD, WE, bE):
    n, d = h.shape
    bm = 2000
    grid = (n // bm,)
    x_spec = pl.BlockSpec((bm, d), lambda i: (i, 0))
    w_spec = pl.BlockSpec((d, d), lambda i: (0, 0))
    b_spec = pl.BlockSpec((1, d), lambda i: (0, 0))

    def body(x_ref, wa, ba, wb, bb, wd, bd, we, be_, oa, ob, od, oe):
        x = x_ref[...]
        oa[...] = jnp.dot(x, wa[...], preferred_element_type=jnp.float32) + ba[...]
        ob[...] = jnp.dot(x, wb[...], preferred_element_type=jnp.float32) + bb[...]
        od[...] = jnp.dot(x, wd[...], preferred_element_type=jnp.float32) + bd[...]
        oe[...] = jnp.dot(x, we[...], preferred_element_type=jnp.float32) + be_[...]

    return pl.pallas_call(
        body,
        grid=grid,
        in_specs=[x_spec, w_spec, b_spec, w_spec, b_spec, w_spec, b_spec,
                  w_spec, b_spec],
        out_specs=[x_spec] * 4,
        out_shape=[jax.ShapeDtypeStruct((n, d), jnp.float32)] * 4,
    )(h, WA, bA.reshape(1, d), WB, bB.reshape(1, d), WD, bD.reshape(1, d),
      WE, bE.reshape(1, d))


def _sc_gather(Dh, Eh, Bh, src, dst):
    n, d = Dh.shape
    ecount = src.shape[0]
    per_w = ecount // _NW
    nchunks = per_w // _K

    @functools.partial(
        pl.kernel,
        mesh=_sc_mesh(),
        out_type=[jax.ShapeDtypeStruct((ecount, d), jnp.float32)] * 2,
        scratch_types=[
            pltpu.VMEM((per_w,), jnp.int32),
            pltpu.VMEM((per_w,), jnp.int32),
        ] + [pltpu.VMEM((_K, d), jnp.float32)] * (3 * _G) + [
            pltpu.SemaphoreType.DMA,
            pltpu.SemaphoreType.DMA,
        ],
    )
    def gk(dh_h, eh_h, bh_h, src_h, dst_h, odeh_h, ob_h, si_all, di_all,
           *bufs_and_sems):
        bufs = bufs_and_sems[:3 * _G]
        sg, sw = bufs_and_sems[3 * _G:]
        rd = bufs[0::3]
        re = bufs[1::3]
        rb = bufs[2::3]
        wid = lax.axis_index("s") * 2 + lax.axis_index("c")
        base = pl.multiple_of(wid * per_w, 8)
        pltpu.sync_copy(src_h.at[pl.ds(base, per_w)], si_all)
        pltpu.sync_copy(dst_h.at[pl.ds(base, per_w)], di_all)

        @pl.loop(0, nchunks // _G)
        def _(jj):
            c0 = jj * _G
            gathers = []
            for par in range(_G):
                off = pl.multiple_of((c0 + par) * _K, 8)
                si = si_all.at[pl.ds(off, _K)]
                di = di_all.at[pl.ds(off, _K)]
                gathers.append(pltpu.async_copy(dh_h.at[si], rd[par], sg))
                gathers.append(pltpu.async_copy(eh_h.at[di], re[par], sg))
                gathers.append(pltpu.async_copy(bh_h.at[si], rb[par], sg))
            for g in gathers:
                g.wait()
            for par in range(_G):
                @pl.loop(0, _K)
                def _(r, par=par):
                    for col in range(0, d, 16):
                        rd[par][r, pl.ds(col, 16)] = (
                            rd[par][r, pl.ds(col, 16)]
                            + re[par][r, pl.ds(col, 16)])
            writes = []
            for par in range(_G):
                b = pl.multiple_of(base + (c0 + par) * _K, 8)
                writes.append(pltpu.async_copy(rd[par], odeh_h.at[pl.ds(b, _K)], sw))
                writes.append(pltpu.async_copy(rb[par], ob_h.at[pl.ds(b, _K)], sw))
            for w in writes:
                w.wait()

    return gk(Dh, Eh, Bh, src, dst)


def _edge_math(e, deh, bsrc, WC, bC):
    m, d = e.shape
    bm = 2000
    grid = (m // bm,)
    x_spec = pl.BlockSpec((bm, d), lambda i: (i, 0))

    def body(e_ref, deh_ref, bs_ref, wc, bc, ms_ref, eo_ref):
        ev = e_ref[...]
        ce = jnp.dot(ev, wc[...], preferred_element_type=jnp.float32) + bc[...]
        e_new = deh_ref[...] + ce
        sig = jax.nn.sigmoid(e_new)
        ms_ref[0] = bs_ref[...] * sig
        ms_ref[1] = sig
        eo_ref[...] = ev + e_new * sig

    return pl.pallas_call(
        body,
        grid=grid,
        in_specs=[x_spec, x_spec, x_spec,
                  pl.BlockSpec((d, d), lambda i: (0, 0)),
                  pl.BlockSpec((1, d), lambda i: (0, 0))],
        out_specs=[pl.BlockSpec((2, bm, d), lambda i: (0, i, 0)), x_spec],
        out_shape=[jax.ShapeDtypeStruct((2, m, d), jnp.float32),
                   jax.ShapeDtypeStruct((m, d), jnp.float32)],
    )(e, deh, bsrc, WC, bC.reshape(1, d))


def _sc_scatter_add(ms_flat, dst, zeros):
    e2, d = ms_flat.shape
    ecount = e2 // 2
    n = zeros.shape[0]
    per_s = ecount // 16
    nchunks = per_s // _K

    @functools.partial(
        pl.kernel,
        mesh=_sc_mesh(),
        out_type=jax.ShapeDtypeStruct((2 * n, d), jnp.float32),
        scratch_types=[
            pltpu.VMEM_SHARED((n, d), jnp.float32),
        ] + [pltpu.VMEM((_K,), jnp.int32)] * _G
          + [pltpu.VMEM((_K, d), jnp.float32)] * _G + [
            pltpu.SemaphoreType.DMA,
            pltpu.SemaphoreType.DMA,
        ],
    )
    def sk(ms_h, dst_h, z_h, o_h, acc, *bufs_and_sems):
        idxs = bufs_and_sems[:_G]
        rows = bufs_and_sems[_G:2 * _G]
        sl, sa = bufs_and_sems[2 * _G:]
        core = lax.axis_index("c")
        sid = lax.axis_index("s")

        @pl.when(sid == 0)
        def _():
            pltpu.sync_copy(z_h, acc)

        plsc.subcore_barrier()
        ebase = core * ecount + sid * per_s
        ibase = sid * per_s

        @pl.loop(0, nchunks // _G)
        def _(jj):
            c0 = jj * _G
            loads = []
            for par in range(_G):
                c = c0 + par
                eoff = pl.multiple_of(ebase + c * _K, 8)
                ioff = pl.multiple_of(ibase + c * _K, 8)
                loads.append(pltpu.async_copy(
                    dst_h.at[pl.ds(ioff, _K)], idxs[par], sl))
                loads.append(pltpu.async_copy(
                    ms_h.at[pl.ds(eoff, _K)], rows[par], sl))
            adds = []
            for par in range(_G):
                loads[2 * par].wait()
                loads[2 * par + 1].wait()
                adds.append(pltpu.async_copy(
                    rows[par], acc.at[idxs[par]], sa, add=True))
            for a in adds:
                a.wait()

        plsc.subcore_barrier()
        nsl = n // 10

        @pl.when(sid < 10)
        def _():
            pltpu.sync_copy(
                acc.at[pl.ds(pl.multiple_of(sid * nsl, 8), nsl)],
                o_h.at[pl.ds(pl.multiple_of(core * n + sid * nsl, 8), nsl)])

    return sk(ms_flat, dst, zeros)


def _node_update(h, Ah, acc):
    n, d = h.shape
    bm = 2000
    grid = (n // bm,)
    x_spec = pl.BlockSpec((bm, d), lambda i: (i, 0))

    def body(h_ref, a_ref, acc_ref, o_ref):
        hn = a_ref[...] + acc_ref[0] / (acc_ref[1] + 1e-6)
        o_ref[...] = h_ref[...] + hn * jax.nn.sigmoid(hn)

    return pl.pallas_call(
        body,
        grid=grid,
        in_specs=[x_spec, x_spec,
                  pl.BlockSpec((2, bm, d), lambda i: (0, i, 0))],
        out_specs=x_spec,
        out_shape=jax.ShapeDtypeStruct((n, d), jnp.float32),
    )(h, Ah, acc)


def kernel(h, e, edge_index, WA, bA, WB, bB, WC, bC, WD, bD, WE, bE):
    src = edge_index[0]
    dst = edge_index[1]
    n, d = h.shape
    ecount = e.shape[0]
    Ah, Bh, Dh, Eh = _node_transforms(h, WA, bA, WB, bB, WD, bD, WE, bE)
    deh, bsrc = _sc_gather(Dh, Eh, Bh, src, dst)
    ms, e_out = _edge_math(e, deh, bsrc, WC, bC)
    zeros = jnp.zeros((n, d), jnp.float32)
    acc_flat = _sc_scatter_add(ms.reshape(2 * ecount, d), dst, zeros)
    h_out = _node_update(h, Ah, acc_flat.reshape(2, n, d))
    return (h_out, e_out)


# speedup vs baseline: 4.4518x; 1.0404x over previous
import functools

import jax
import jax.numpy as jnp
from jax import lax
from jax.experimental import pallas as pl
from jax.experimental.pallas import tpu as pltpu
from jax.experimental.pallas import tpu_sc as plsc


def _sc_mesh():
    return plsc.VectorSubcoreMesh(core_axis_name="c", subcore_axis_name="s")


_NW = 32
_K = 40
_G = 5


def _node_transforms(h, WA, bA, WB, bB, WD, bD, WE, bE):
    n, d = h.shape
    bm = 2000
    grid = (n // bm,)
    x_spec = pl.BlockSpec((bm, d), lambda i: (i, 0))
    w_spec = pl.BlockSpec((d, d), lambda i: (0, 0))
    b_spec = pl.BlockSpec((1, d), lambda i: (0, 0))

    def body(x_ref, wa, ba, wb, bb, wd, bd, we, be_, oa, ob, od, oe):
        x = x_ref[...]
        oa[...] = jnp.dot(x, wa[...], preferred_element_type=jnp.float32) + ba[...]
        ob[...] = jnp.dot(x, wb[...], preferred_element_type=jnp.float32) + bb[...]
        od[...] = jnp.dot(x, wd[...], preferred_element_type=jnp.float32) + bd[...]
        oe[...] = jnp.dot(x, we[...], preferred_element_type=jnp.float32) + be_[...]

    return pl.pallas_call(
        body,
        grid=grid,
        in_specs=[x_spec, w_spec, b_spec, w_spec, b_spec, w_spec, b_spec,
                  w_spec, b_spec],
        out_specs=[x_spec] * 4,
        out_shape=[jax.ShapeDtypeStruct((n, d), jnp.float32)] * 4,
    )(h, WA, bA.reshape(1, d), WB, bB.reshape(1, d), WD, bD.reshape(1, d),
      WE, bE.reshape(1, d))


def _sc_gather(Dh, Eh, Bh, src, dst, off, cnt):
    n, d = Dh.shape
    per_w = cnt // _NW
    nchunks = per_w // _K

    @functools.partial(
        pl.kernel,
        mesh=_sc_mesh(),
        out_type=[jax.ShapeDtypeStruct((cnt, d), jnp.float32)] * 2,
        scratch_types=[
            pltpu.VMEM((per_w,), jnp.int32),
            pltpu.VMEM((per_w,), jnp.int32),
        ] + [pltpu.VMEM((_K, d), jnp.float32)] * (3 * _G) + [
            pltpu.SemaphoreType.DMA,
            pltpu.SemaphoreType.DMA,
        ],
    )
    def gk(dh_h, eh_h, bh_h, src_h, dst_h, odeh_h, ob_h, si_all, di_all,
           *bufs_and_sems):
        bufs = bufs_and_sems[:3 * _G]
        sg, sw = bufs_and_sems[3 * _G:]
        rd = bufs[0::3]
        re = bufs[1::3]
        rb = bufs[2::3]
        wid = lax.axis_index("s") * 2 + lax.axis_index("c")
        obase = pl.multiple_of(wid * per_w, 8)
        base = pl.multiple_of(off + wid * per_w, 8)
        pltpu.sync_copy(src_h.at[pl.ds(base, per_w)], si_all)
        pltpu.sync_copy(dst_h.at[pl.ds(base, per_w)], di_all)

        @pl.loop(0, nchunks // _G)
        def _(jj):
            c0 = jj * _G
            gathers = []
            for par in range(_G):
                off = pl.multiple_of((c0 + par) * _K, 8)
                si = si_all.at[pl.ds(off, _K)]
                di = di_all.at[pl.ds(off, _K)]
                gathers.append(pltpu.async_copy(dh_h.at[si], rd[par], sg))
                gathers.append(pltpu.async_copy(eh_h.at[di], re[par], sg))
                gathers.append(pltpu.async_copy(bh_h.at[si], rb[par], sg))
            for g in gathers:
                g.wait()
            for par in range(_G):
                @pl.loop(0, _K)
                def _(r, par=par):
                    for col in range(0, d, 16):
                        rd[par][r, pl.ds(col, 16)] = (
                            rd[par][r, pl.ds(col, 16)]
                            + re[par][r, pl.ds(col, 16)])
            writes = []
            for par in range(_G):
                b = pl.multiple_of(obase + (c0 + par) * _K, 8)
                writes.append(pltpu.async_copy(rd[par], odeh_h.at[pl.ds(b, _K)], sw))
                writes.append(pltpu.async_copy(rb[par], ob_h.at[pl.ds(b, _K)], sw))
            for w in writes:
                w.wait()

    return gk(Dh, Eh, Bh, src, dst)


def _edge_math(e, deh, bsrc, WC, bC, blk_off):
    cnt, d = deh.shape
    bm = 2000
    grid = (cnt // bm,)
    x_spec = pl.BlockSpec((bm, d), lambda i: (i, 0))
    e_spec = pl.BlockSpec((bm, d), lambda i: (i + blk_off, 0))

    def body(e_ref, deh_ref, bs_ref, wc, bc, ms_ref, eo_ref):
        ev = e_ref[...]
        ce = jnp.dot(ev, wc[...], preferred_element_type=jnp.float32) + bc[...]
        e_new = deh_ref[...] + ce
        sig = jax.nn.sigmoid(e_new)
        ms_ref[0] = bs_ref[...] * sig
        ms_ref[1] = sig
        eo_ref[...] = ev + e_new * sig

    return pl.pallas_call(
        body,
        grid=grid,
        in_specs=[e_spec, x_spec, x_spec,
                  pl.BlockSpec((d, d), lambda i: (0, 0)),
                  pl.BlockSpec((1, d), lambda i: (0, 0))],
        out_specs=[pl.BlockSpec((2, bm, d), lambda i: (0, i, 0)), x_spec],
        out_shape=[jax.ShapeDtypeStruct((2, cnt, d), jnp.float32),
                   jax.ShapeDtypeStruct((cnt, d), jnp.float32)],
    )(e, deh, bsrc, WC, bC.reshape(1, d))


def _sc_scatter_add(ms_flat, dst, zeros, off):
    e2, d = ms_flat.shape
    cnt = e2 // 2
    n = zeros.shape[0]
    per_s = cnt // 16
    nchunks = per_s // _K

    @functools.partial(
        pl.kernel,
        mesh=_sc_mesh(),
        out_type=jax.ShapeDtypeStruct((2 * n, d), jnp.float32),
        scratch_types=[
            pltpu.VMEM_SHARED((n, d), jnp.float32),
        ] + [pltpu.VMEM((_K,), jnp.int32)] * _G
          + [pltpu.VMEM((_K, d), jnp.float32)] * _G + [
            pltpu.SemaphoreType.DMA,
            pltpu.SemaphoreType.DMA,
        ],
    )
    def sk(ms_h, dst_h, z_h, o_h, acc, *bufs_and_sems):
        idxs = bufs_and_sems[:_G]
        rows = bufs_and_sems[_G:2 * _G]
        sl, sa = bufs_and_sems[2 * _G:]
        core = lax.axis_index("c")
        sid = lax.axis_index("s")

        @pl.when(sid == 0)
        def _():
            pltpu.sync_copy(z_h, acc)

        plsc.subcore_barrier()
        ebase = core * cnt + sid * per_s
        ibase = off + sid * per_s

        @pl.loop(0, nchunks // _G)
        def _(jj):
            c0 = jj * _G
            loads = []
            for par in range(_G):
                c = c0 + par
                eoff = pl.multiple_of(ebase + c * _K, 8)
                ioff = pl.multiple_of(ibase + c * _K, 8)
                loads.append(pltpu.async_copy(
                    dst_h.at[pl.ds(ioff, _K)], idxs[par], sl))
                loads.append(pltpu.async_copy(
                    ms_h.at[pl.ds(eoff, _K)], rows[par], sl))
            adds = []
            for par in range(_G):
                loads[2 * par].wait()
                loads[2 * par + 1].wait()
                adds.append(pltpu.async_copy(
                    rows[par], acc.at[idxs[par]], sa, add=True))
            for a in adds:
                a.wait()

        plsc.subcore_barrier()
        nsl = n // 10

        @pl.when(sid < 10)
        def _():
            pltpu.sync_copy(
                acc.at[pl.ds(pl.multiple_of(sid * nsl, 8), nsl)],
                o_h.at[pl.ds(pl.multiple_of(core * n + sid * nsl, 8), nsl)])

    return sk(ms_flat, dst, zeros)


def _node_update(h, Ah, acc1, acc2):
    n, d = h.shape
    bm = 2000
    grid = (n // bm,)
    x_spec = pl.BlockSpec((bm, d), lambda i: (i, 0))
    a2_spec = pl.BlockSpec((2, bm, d), lambda i: (0, i, 0))

    def body(h_ref, a_ref, c1_ref, c2_ref, o_ref):
        num = c1_ref[0] + c2_ref[0]
        den = c1_ref[1] + c2_ref[1] + 1e-6
        hn = a_ref[...] + num / den
        o_ref[...] = h_ref[...] + hn * jax.nn.sigmoid(hn)

    return pl.pallas_call(
        body,
        grid=grid,
        in_specs=[x_spec, x_spec, a2_spec, a2_spec],
        out_specs=x_spec,
        out_shape=jax.ShapeDtypeStruct((n, d), jnp.float32),
    )(h, Ah, acc1, acc2)


def kernel(h, e, edge_index, WA, bA, WB, bB, WC, bC, WD, bD, WE, bE):
    src = edge_index[0]
    dst = edge_index[1]
    n, d = h.shape
    ecount = e.shape[0]
    half = ecount // 2
    Ah, Bh, Dh, Eh = _node_transforms(h, WA, bA, WB, bB, WD, bD, WE, bE)
    zeros = jnp.zeros((n, d), jnp.float32)
    deh1, bsrc1 = _sc_gather(Dh, Eh, Bh, src, dst, 0, half)
    deh2, bsrc2 = _sc_gather(Dh, Eh, Bh, src, dst, half, half)
    ms1, eo1 = _edge_math(e, deh1, bsrc1, WC, bC, 0)
    ms2, eo2 = _edge_math(e, deh2, bsrc2, WC, bC, half // 2000)
    acc1 = _sc_scatter_add(ms1.reshape(2 * half, d), dst, zeros, 0)
    acc2 = _sc_scatter_add(ms2.reshape(2 * half, d), dst, zeros, half)
    e_out = jnp.concatenate([eo1, eo2], axis=0)
    h_out = _node_update(h, Ah, acc1.reshape(2, n, d), acc2.reshape(2, n, d))
    return (h_out, e_out)
